# Initial kernel scaffold; baseline (speedup 1.0000x reference)
import jax
import jax.numpy as jnp
from jax.experimental import pallas as pl


def kernel(src, index, dim_size):
    raise NotImplementedError("write your pallas kernel here")


# speedup vs baseline: 1.4707x; 1.4707x over previous
import jax
import jax.numpy as jnp
from jax import lax
from jax.experimental import pallas as pl
from jax.experimental.pallas import tpu as pltpu
from jax.experimental.pallas import tpu_sc as plsc

N_NODES = 10000
D = 128
NW = 32
NPT = 320
BLK = 32


def _sc_body(src_hbm, idx_hbm, bounds_hbm, bias_hbm, out_hbm,
             bounds_v, bias_v, idx_b0, idx_b1, src_b0, src_b1, stage,
             sem_i0, sem_i1, sem_s0, sem_s1):
    c = lax.axis_index("c")
    s = lax.axis_index("s")
    w = s * 2 + c

    pltpu.sync_copy(bounds_hbm, bounds_v)
    pltpu.sync_copy(bias_hbm, bias_v)

    n0 = jnp.minimum(w * NPT, N_NODES - NPT)

    gather_dnums = lax.GatherDimensionNumbers(
        offset_dims=(), collapsed_slice_dims=(0,), start_index_map=(0,))

    def dyn_gather(vec, idxvec):
        return lax.gather(vec, idxvec[:, None], gather_dnums, (1,),
                          mode=lax.GatherScatterMode.PROMISE_IN_BOUNDS)

    wv = jnp.zeros((16,), jnp.int32) + w

    def read_bound(base):
        v0 = bounds_v[base:base + 16]
        v1 = bounds_v[base + 16:base + 32]
        sel = jnp.where(w < 16,
                        dyn_gather(v0, jnp.clip(wv, 0, 15)),
                        dyn_gather(v1, jnp.clip(wv - 16, 0, 15)))
        return sel[0]

    e_lo = read_bound(0)
    e_hi = read_bound(NW)

    zeros = jnp.zeros((16,), jnp.float32)
    ninf = jnp.full((16,), -jnp.inf, jnp.float32)

    def init_row(r, carry):
        for k in range(8):
            stage[r, 16 * k:16 * (k + 1)] = zeros
            stage[r, 128 + 16 * k:128 + 16 * (k + 1)] = ninf
        return carry

    lax.fori_loop(0, NPT, init_row, 0)

    iota = lax.iota(jnp.int32, 16)
    cols = [iota + 16 * k for k in range(8)]
    colsm = [iota + 128 + 16 * k for k in range(8)]

    def bcast_lane(vec, j):
        cj = jnp.full((16, 1), j, jnp.int32)
        return lax.gather(vec, cj, gather_dnums, (1,),
                          mode=lax.GatherScatterMode.PROMISE_IN_BOUNDS)

    def process_group(idxbuf, srcbuf, e_base, goff, carry):
        prev, accs, accm = carry
        idxv = idxbuf[goff:goff + 16]
        rowv = idxv - n0
        ev = e_base + iota
        m_i32 = jnp.where((ev >= e_lo) & (ev < e_hi), 1, 0)
        for j in range(16):
            rsp = bcast_lane(rowv, j)
            mvb = bcast_lane(m_i32, j) != 0
            same = rsp == prev
            new_s, new_m = [], []
            for k in range(8):
                v = srcbuf[goff + j, 16 * k:16 * (k + 1)]
                sv = jnp.where(same, accs[k], zeros) + v
                mv = jnp.maximum(jnp.where(same, accm[k], ninf), v)
                plsc.store_scatter(stage, [rsp, cols[k]], sv, mask=mvb)
                plsc.store_scatter(stage, [rsp, colsm[k]], mv, mask=mvb)
                new_s.append(sv)
                new_m.append(mv)
            accs, accm, prev = tuple(new_s), tuple(new_m), rsp
        return prev, accs, accm

    def start(eb, b):
        ib, sb = (idx_b0, src_b0) if b == 0 else (idx_b1, src_b1)
        si, ss = (sem_i0, sem_s0) if b == 0 else (sem_i1, sem_s1)
        pltpu.make_async_copy(idx_hbm.at[pl.ds(eb, BLK)], ib, si).start()
        pltpu.make_async_copy(src_hbm.at[pl.ds(eb, BLK), :], sb, ss).start()

    def wait(b):
        ib, sb = (idx_b0, src_b0) if b == 0 else (idx_b1, src_b1)
        si, ss = (sem_i0, sem_s0) if b == 0 else (sem_i1, sem_s1)
        pltpu.make_async_copy(idx_hbm.at[pl.ds(0, BLK)], ib, si).wait()
        pltpu.make_async_copy(src_hbm.at[pl.ds(0, BLK), :], sb, ss).wait()

    e0 = (e_lo // 16) * 16
    e1 = ((e_hi + 15) // 16) * 16

    main_len = ((e1 - e0) // (2 * BLK)) * (2 * BLK)
    nbp = main_len // (2 * BLK)

    carry0 = (jnp.full((16,), -1, jnp.int32),
              tuple(zeros for _ in range(8)),
              tuple(ninf for _ in range(8)))

    @pl.when(nbp > 0)
    def _():
        start(e0, 0)

    def pair_body(p, carry):
        eb0 = e0 + p * (2 * BLK)
        start(eb0 + BLK, 1)
        wait(0)
        for g in range(BLK // 16):
            carry = process_group(idx_b0, src_b0, eb0 + 16 * g, 16 * g, carry)

        @pl.when(p + 1 < nbp)
        def _():
            start(eb0 + 2 * BLK, 0)

        wait(1)
        for g in range(BLK // 16):
            carry = process_group(idx_b1, src_b1, eb0 + BLK + 16 * g, 16 * g,
                                  carry)
        return carry

    carry = lax.fori_loop(0, nbp, pair_body, carry0)

    t0 = e0 + main_len
    nt = (e1 - t0) // 16

    def tail_body(t, carry):
        eb = t0 + t * 16
        pltpu.sync_copy(idx_hbm.at[pl.ds(eb, 16)], idx_b0.at[pl.ds(0, 16)])
        pltpu.sync_copy(src_hbm.at[pl.ds(eb, 16), :],
                        src_b0.at[pl.ds(0, 16), :])
        return process_group(idx_b0, src_b0, eb, 0, carry)

    lax.fori_loop(0, nt, tail_body, carry)

    bias = bias_v[0:16]

    def fix_row(r, carry):
        for k in range(8):
            sv = stage[r, 16 * k:16 * (k + 1)]
            stage[r, 16 * k:16 * (k + 1)] = sv + bias
            mv = stage[r, 128 + 16 * k:128 + 16 * (k + 1)]
            mv = jnp.where(mv == -jnp.inf, zeros, mv) + bias
            stage[r, 128 + 16 * k:128 + 16 * (k + 1)] = mv
        return carry

    lax.fori_loop(0, NPT, fix_row, 0)

    pltpu.sync_copy(stage, out_hbm.at[pl.ds(n0, NPT), :])


def kernel(src, index, dim_size):
    index = index.astype(jnp.int32)
    n0_arr = jnp.minimum(jnp.arange(NW, dtype=jnp.int32) * NPT, N_NODES - NPT)
    lo = jnp.searchsorted(index, n0_arr, side="left")
    hi = jnp.searchsorted(index, n0_arr + NPT, side="left")
    bounds = jnp.concatenate([lo, hi]).astype(jnp.int32)
    bias_val = (jnp.asarray(dim_size, jnp.int32) - N_NODES).astype(jnp.float32)
    bias = jnp.zeros((16,), jnp.float32) + bias_val

    mesh = plsc.VectorSubcoreMesh(core_axis_name="c", subcore_axis_name="s")
    out = pl.kernel(
        _sc_body,
        out_type=jax.ShapeDtypeStruct((N_NODES, 2 * D), jnp.float32),
        mesh=mesh,
        compiler_params=pltpu.CompilerParams(use_tc_tiling_on_sc=False,
                                             needs_layout_passes=False),
        scratch_types=[
            pltpu.VMEM((2 * NW,), jnp.int32),
            pltpu.VMEM((16,), jnp.float32),
            pltpu.VMEM((BLK,), jnp.int32),
            pltpu.VMEM((BLK,), jnp.int32),
            pltpu.VMEM((BLK, D), jnp.float32),
            pltpu.VMEM((BLK, D), jnp.float32),
            pltpu.VMEM((NPT, 2 * D), jnp.float32),
            pltpu.SemaphoreType.DMA,
            pltpu.SemaphoreType.DMA,
            pltpu.SemaphoreType.DMA,
            pltpu.SemaphoreType.DMA,
        ],
    )(src, index, bounds, bias)
    return out


# speedup vs baseline: 3.2440x; 2.2058x over previous
import jax
import jax.numpy as jnp
from jax import lax
from jax.experimental import pallas as pl
from jax.experimental.pallas import tpu as pltpu
from jax.experimental.pallas import tpu_sc as plsc

N_NODES = 10000
D = 128
NW = 32
NPT = 320
BLK = 128


def _sc_body(src_hbm, idx_hbm, bounds_hbm, bias_hbm, out_hbm,
             bounds_v, bias_v, idx_b0, idx_b1, src_b0, src_b1, stage,
             sem_i0, sem_i1, sem_s0, sem_s1):
    c = lax.axis_index("c")
    s = lax.axis_index("s")
    w = s * 2 + c

    pltpu.sync_copy(bounds_hbm, bounds_v)
    pltpu.sync_copy(bias_hbm, bias_v)

    n0 = jnp.minimum(w * NPT, N_NODES - NPT)

    gather_dnums = lax.GatherDimensionNumbers(
        offset_dims=(), collapsed_slice_dims=(0,), start_index_map=(0,))

    def dyn_gather(vec, idxvec):
        return lax.gather(vec, idxvec[:, None], gather_dnums, (1,),
                          mode=lax.GatherScatterMode.PROMISE_IN_BOUNDS)

    wv = jnp.zeros((16,), jnp.int32) + w

    def read_bound(base):
        v0 = bounds_v[base:base + 16]
        v1 = bounds_v[base + 16:base + 32]
        sel = jnp.where(w < 16,
                        dyn_gather(v0, jnp.clip(wv, 0, 15)),
                        dyn_gather(v1, jnp.clip(wv - 16, 0, 15)))
        return sel[0]

    e_lo = read_bound(0)
    e_hi = read_bound(NW)

    zeros = jnp.zeros((16,), jnp.float32)
    ninf = jnp.full((16,), -jnp.inf, jnp.float32)

    def init_row(r, carry):
        for k in range(8):
            stage[r, 16 * k:16 * (k + 1)] = zeros
            stage[r, 128 + 16 * k:128 + 16 * (k + 1)] = ninf
        return carry

    lax.fori_loop(0, NPT, init_row, 0)

    iota = lax.iota(jnp.int32, 16)
    cols = [iota + 16 * k for k in range(8)]
    colsm = [iota + 128 + 16 * k for k in range(8)]

    def bcast_lane(vec, j):
        cj = jnp.full((16, 1), j, jnp.int32)
        return lax.gather(vec, cj, gather_dnums, (1,),
                          mode=lax.GatherScatterMode.PROMISE_IN_BOUNDS)

    def process_group(idxbuf, srcbuf, e_base, goff, carry):
        prev, accs, accm = carry
        idxv = idxbuf[pl.ds(goff, 16)]
        rowv = idxv - n0
        ev = e_base + iota
        m_i32 = jnp.where((ev >= e_lo) & (ev < e_hi), 1, 0)
        for j in range(16):
            rsp = bcast_lane(rowv, j)
            mvb = bcast_lane(m_i32, j) != 0
            same = rsp == prev
            new_s, new_m = [], []
            for k in range(8):
                v = srcbuf[goff + j, 16 * k:16 * (k + 1)]
                sv = jnp.where(same, accs[k], zeros) + v
                mv = jnp.maximum(jnp.where(same, accm[k], ninf), v)
                plsc.store_scatter(stage, [rsp, cols[k]], sv, mask=mvb)
                plsc.store_scatter(stage, [rsp, colsm[k]], mv, mask=mvb)
                new_s.append(sv)
                new_m.append(mv)
            accs, accm, prev = tuple(new_s), tuple(new_m), rsp
        return prev, accs, accm

    def start(eb, b):
        ib, sb = (idx_b0, src_b0) if b == 0 else (idx_b1, src_b1)
        si, ss = (sem_i0, sem_s0) if b == 0 else (sem_i1, sem_s1)
        pltpu.make_async_copy(idx_hbm.at[pl.ds(eb, BLK)], ib, si).start()
        pltpu.make_async_copy(src_hbm.at[pl.ds(eb, BLK), :], sb, ss).start()

    def wait(b):
        ib, sb = (idx_b0, src_b0) if b == 0 else (idx_b1, src_b1)
        si, ss = (sem_i0, sem_s0) if b == 0 else (sem_i1, sem_s1)
        pltpu.make_async_copy(idx_hbm.at[pl.ds(0, BLK)], ib, si).wait()
        pltpu.make_async_copy(src_hbm.at[pl.ds(0, BLK), :], sb, ss).wait()

    e0 = (e_lo // 16) * 16
    e1 = ((e_hi + 15) // 16) * 16

    main_len = ((e1 - e0) // (2 * BLK)) * (2 * BLK)
    nbp = main_len // (2 * BLK)

    carry0 = (jnp.full((16,), -1, jnp.int32),
              tuple(zeros for _ in range(8)),
              tuple(ninf for _ in range(8)))

    @pl.when(nbp > 0)
    def _():
        start(e0, 0)

    def pair_body(p, carry):
        eb0 = e0 + p * (2 * BLK)
        start(eb0 + BLK, 1)
        wait(0)

        def grp0(g, c):
            return process_group(idx_b0, src_b0, eb0 + 16 * g, 16 * g, c)

        carry = lax.fori_loop(0, BLK // 16, grp0, carry)

        @pl.when(p + 1 < nbp)
        def _():
            start(eb0 + 2 * BLK, 0)

        wait(1)

        def grp1(g, c):
            return process_group(idx_b1, src_b1, eb0 + BLK + 16 * g, 16 * g, c)

        carry = lax.fori_loop(0, BLK // 16, grp1, carry)
        return carry

    carry = lax.fori_loop(0, nbp, pair_body, carry0)

    t0 = e0 + main_len
    nt = (e1 - t0) // 16

    def tail_body(t, carry):
        eb = t0 + t * 16
        pltpu.sync_copy(idx_hbm.at[pl.ds(eb, 16)], idx_b0.at[pl.ds(0, 16)])
        pltpu.sync_copy(src_hbm.at[pl.ds(eb, 16), :],
                        src_b0.at[pl.ds(0, 16), :])
        return process_group(idx_b0, src_b0, eb, 0, carry)

    lax.fori_loop(0, nt, tail_body, carry)

    bias = bias_v[0:16]

    def fix_row(r, carry):
        for k in range(8):
            sv = stage[r, 16 * k:16 * (k + 1)]
            stage[r, 16 * k:16 * (k + 1)] = sv + bias
            mv = stage[r, 128 + 16 * k:128 + 16 * (k + 1)]
            mv = jnp.where(mv == -jnp.inf, zeros, mv) + bias
            stage[r, 128 + 16 * k:128 + 16 * (k + 1)] = mv
        return carry

    lax.fori_loop(0, NPT, fix_row, 0)

    pltpu.sync_copy(stage, out_hbm.at[pl.ds(n0, NPT), :])


def kernel(src, index, dim_size):
    index = index.astype(jnp.int32)
    n0_arr = jnp.minimum(jnp.arange(NW, dtype=jnp.int32) * NPT, N_NODES - NPT)
    lo = jnp.searchsorted(index, n0_arr, side="left")
    hi = jnp.searchsorted(index, n0_arr + NPT, side="left")
    bounds = jnp.concatenate([lo, hi]).astype(jnp.int32)
    bias_val = (jnp.asarray(dim_size, jnp.int32) - N_NODES).astype(jnp.float32)
    bias = jnp.zeros((16,), jnp.float32) + bias_val

    mesh = plsc.VectorSubcoreMesh(core_axis_name="c", subcore_axis_name="s")
    out = pl.kernel(
        _sc_body,
        out_type=jax.ShapeDtypeStruct((N_NODES, 2 * D), jnp.float32),
        mesh=mesh,
        compiler_params=pltpu.CompilerParams(use_tc_tiling_on_sc=False,
                                             needs_layout_passes=False),
        scratch_types=[
            pltpu.VMEM((2 * NW,), jnp.int32),
            pltpu.VMEM((16,), jnp.float32),
            pltpu.VMEM((BLK,), jnp.int32),
            pltpu.VMEM((BLK,), jnp.int32),
            pltpu.VMEM((BLK, D), jnp.float32),
            pltpu.VMEM((BLK, D), jnp.float32),
            pltpu.VMEM((NPT, 2 * D), jnp.float32),
            pltpu.SemaphoreType.DMA,
            pltpu.SemaphoreType.DMA,
            pltpu.SemaphoreType.DMA,
            pltpu.SemaphoreType.DMA,
        ],
    )(src, index, bounds, bias)
    return out
